# Initial kernel scaffold; baseline (speedup 1.0000x reference)
import functools

import jax
import jax.numpy as jnp
from jax import lax
from jax.experimental import pallas as pl
from jax.experimental.pallas import tpu as pltpu
from jax.experimental.pallas import tpu_sc as plsc

VOCAB = 128
D = 64
BATCH = 4096
SEQ = 200
TOTAL = BATCH * SEQ
NC = 2
NS = 16
NW = NC * NS
ROWS_PER_W = TOTAL // NW
CHUNK = 128
K = 8
STEP_ROWS = CHUNK * K
N_STEPS = ROWS_PER_W // STEP_ROWS
IDX_ROWS_PER_W = ROWS_PER_W // CHUNK


def _sc_gather(idx2d, table):
    mesh = plsc.VectorSubcoreMesh(core_axis_name="c", subcore_axis_name="s")

    @functools.partial(
        pl.kernel,
        mesh=mesh,
        out_type=jax.ShapeDtypeStruct((TOTAL, D), jnp.float32),
        scratch_types=[
            pltpu.VMEM((K, CHUNK), jnp.int32),
            pltpu.VMEM((STEP_ROWS, D), jnp.float32),
            pltpu.SemaphoreType.DMA,
        ],
    )
    def k(idx_hbm, table_hbm, out_hbm, idx_v, rows_v, sem):
        wid = lax.axis_index("s") * NC + lax.axis_index("c")
        row0 = wid * IDX_ROWS_PER_W

        def step(i, carry):
            r = row0 + i * K
            pltpu.sync_copy(idx_hbm.at[pl.ds(r, K)], idx_v)
            copies = [
                pltpu.async_copy(
                    table_hbm.at[idx_v.at[j]],
                    rows_v.at[pl.ds(j * CHUNK, CHUNK)],
                    sem,
                )
                for j in range(K)
            ]
            for c in copies:
                c.wait()
            pltpu.sync_copy(rows_v, out_hbm.at[pl.ds(r * CHUNK, STEP_ROWS)])
            return carry

        lax.fori_loop(0, N_STEPS, step, 0)

    return k(idx2d, table)


def kernel(smiles_indices, embedding_table):
    idx2d = smiles_indices.astype(jnp.int32).reshape(TOTAL // CHUNK, CHUNK)
    out = _sc_gather(idx2d, embedding_table)
    return out.reshape(BATCH, SEQ, D)


# speedup vs baseline: 2.6858x; 2.6858x over previous
import functools

import jax
import jax.numpy as jnp
from jax import lax
from jax.experimental import pallas as pl
from jax.experimental.pallas import tpu as pltpu
from jax.experimental.pallas import tpu_sc as plsc

VOCAB = 128
D = 64
BATCH = 4096
SEQ = 200
TOTAL = BATCH * SEQ
NC = 2
NS = 16
NW = NC * NS
ROWS_PER_W = TOTAL // NW
CHUNK = 128
K = 8
STEP_ROWS = CHUNK * K
N_STEPS = ROWS_PER_W // STEP_ROWS
IDX_ROWS_PER_W = ROWS_PER_W // CHUNK


def _sc_gather(idx2d, table):
    mesh = plsc.VectorSubcoreMesh(core_axis_name="c", subcore_axis_name="s")

    @functools.partial(
        pl.kernel,
        mesh=mesh,
        out_type=jax.ShapeDtypeStruct((TOTAL, D), jnp.float32),
        scratch_types=[
            pltpu.VMEM((K, CHUNK), jnp.int32),
            pltpu.VMEM((STEP_ROWS, D), jnp.float32),
            pltpu.SemaphoreType.DMA,
        ],
        compiler_params=pltpu.CompilerParams(use_tc_tiling_on_sc=False),
    )
    def k(idx_hbm, table_hbm, out_hbm, idx_v, rows_v, sem):
        wid = lax.axis_index("s") * NC + lax.axis_index("c")
        row0 = wid * IDX_ROWS_PER_W

        def step(i, carry):
            r = row0 + i * K
            pltpu.sync_copy(idx_hbm.at[pl.ds(r, K)], idx_v)
            copies = [
                pltpu.async_copy(
                    table_hbm.at[idx_v.at[j]],
                    rows_v.at[pl.ds(j * CHUNK, CHUNK)],
                    sem,
                )
                for j in range(K)
            ]
            for c in copies:
                c.wait()
            pltpu.sync_copy(rows_v, out_hbm.at[pl.ds(r * CHUNK, STEP_ROWS)])
            return carry

        lax.fori_loop(0, N_STEPS, step, 0)

    return k(idx2d, table)


def kernel(smiles_indices, embedding_table):
    idx2d = smiles_indices.astype(jnp.int32).reshape(TOTAL // CHUNK, CHUNK)
    out = _sc_gather(idx2d, embedding_table)
    return out.reshape(BATCH, SEQ, D)


# speedup vs baseline: 5.0065x; 1.8641x over previous
import functools

import jax
import jax.numpy as jnp
from jax import lax
from jax.experimental import pallas as pl
from jax.experimental.pallas import tpu as pltpu
from jax.experimental.pallas import tpu_sc as plsc

VOCAB = 128
D = 64
BATCH = 4096
SEQ = 200
TOTAL = BATCH * SEQ
NC = 2
NS = 16
NW = NC * NS
ROWS_PER_W = TOTAL // NW
CHUNK = 128
K = 5
STEP_ROWS = CHUNK * K
N_STEPS = ROWS_PER_W // STEP_ROWS
IDX_ROWS_PER_W = ROWS_PER_W // CHUNK


def _sc_gather(idx2d, table):
    mesh = plsc.VectorSubcoreMesh(core_axis_name="c", subcore_axis_name="s")

    @functools.partial(
        pl.kernel,
        mesh=mesh,
        out_type=jax.ShapeDtypeStruct((TOTAL, D), jnp.float32),
        scratch_types=[
            pltpu.VMEM((IDX_ROWS_PER_W, CHUNK), jnp.int32),
            pltpu.VMEM((2, STEP_ROWS, D), jnp.float32),
            pltpu.VMEM((VOCAB, D), jnp.float32),
            pltpu.VMEM_SHARED((VOCAB, D), jnp.float32),
            pltpu.SemaphoreType.DMA,
            pltpu.SemaphoreType.DMA,
        ],
        compiler_params=pltpu.CompilerParams(use_tc_tiling_on_sc=False),
    )
    def k(idx_hbm, table_hbm, out_hbm, idx_v, rows_v, tab_v, tab_sh, sem_g,
          sem_o):
        cid = lax.axis_index("c")
        sid = lax.axis_index("s")
        wid = sid * NC + cid
        row0 = wid * IDX_ROWS_PER_W

        @pl.when(sid == 0)
        def _():
            pltpu.sync_copy(table_hbm, tab_v)
            pltpu.sync_copy(tab_v, tab_sh)

        plsc.subcore_barrier()

        pltpu.sync_copy(idx_hbm.at[pl.ds(row0, IDX_ROWS_PER_W)], idx_v)

        def fire_gathers(i, slot):
            for j in range(K):
                pltpu.async_copy(
                    tab_sh.at[idx_v.at[i * K + j]],
                    rows_v.at[slot].at[pl.ds(j * CHUNK, CHUNK)],
                    sem_g,
                )

        def wait_gathers(slot):
            pltpu.make_async_copy(
                out_hbm.at[pl.ds(0, STEP_ROWS)], rows_v.at[slot], sem_g
            ).wait()

        def fire_put(i, slot):
            pltpu.async_copy(
                rows_v.at[slot],
                out_hbm.at[pl.ds((row0 + i * K) * CHUNK, STEP_ROWS)],
                sem_o,
            )

        def wait_put():
            pltpu.make_async_copy(
                rows_v.at[0], out_hbm.at[pl.ds(0, STEP_ROWS)], sem_o
            ).wait()

        fire_gathers(0, 0)

        def step(i, carry):
            slot = lax.rem(i, 2)
            wait_gathers(slot)
            fire_put(i, slot)

            @pl.when(jnp.logical_and(i >= 1, i + 1 < N_STEPS))
            def _():
                wait_put()

            @pl.when(i + 1 < N_STEPS)
            def _():
                fire_gathers(i + 1, 1 - slot)

            return carry

        lax.fori_loop(0, N_STEPS, step, 0)
        wait_put()
        wait_put()

    return k(idx2d, table)


def kernel(smiles_indices, embedding_table):
    idx2d = smiles_indices.astype(jnp.int32).reshape(TOTAL // CHUNK, CHUNK)
    out = _sc_gather(idx2d, embedding_table)
    return out.reshape(BATCH, SEQ, D)
